# baseline (speedup 1.0000x reference)
import functools

import jax
import jax.numpy as jnp
from jax import lax
from jax.experimental import pallas as pl
from jax.experimental.pallas import tpu as pltpu
from jax.experimental.pallas import tpu_sc as plsc

N = 3_200_000
S = 8192
NC, NS, L = 2, 16, 16
NW = NC * NS
R = N // NW
C = 4_000
NCHUNK = R // C
G = C // L
SEG = S // NS


_GATHER_DNUMS = lax.GatherDimensionNumbers(
    offset_dims=(), collapsed_slice_dims=(0,), start_index_map=(0,))


def _shift_up(x, iota):
    idx = jnp.minimum(iota + 1, L - 1)
    return lax.gather(x, idx[:, None], _GATHER_DNUMS, slice_sizes=(1,),
                      mode=lax.GatherScatterMode.PROMISE_IN_BOUNDS)


def _sc_body(p_hbm, t_hbm, ids_hbm, out_hbm,
             p_buf, t_buf, id_buf, acc, shared, colbuf):
    c = lax.axis_index("c")
    s = lax.axis_index("s")
    wid = c * NS + s
    row0 = wid * R

    iota = lax.iota(jnp.int32, L)
    col3 = iota * 3
    zeros = jnp.zeros((L,), jnp.float32)

    def zero_body(i, carry):
        acc[pl.ds(i * L, L)] = zeros
        return carry
    lax.fori_loop(0, S // L, zero_body, 0)

    def chunk_body(k, carry):
        base = row0 + k * C
        pltpu.sync_copy(p_hbm.at[pl.ds(base * 3, C * 3)], p_buf)
        pltpu.sync_copy(t_hbm.at[pl.ds(base * 3, C * 3)], t_buf)
        pltpu.sync_copy(ids_hbm.at[pl.ds(base, C)], id_buf)

        def group_body(g, gcarry):
            b3 = g * (L * 3)
            idx0 = b3 + col3
            idx1 = idx0 + 1
            idx2 = idx0 + 2
            d0 = plsc.load_gather(p_buf, [idx0]) - plsc.load_gather(t_buf, [idx0])
            d1 = plsc.load_gather(p_buf, [idx1]) - plsc.load_gather(t_buf, [idx1])
            d2 = plsc.load_gather(p_buf, [idx2]) - plsc.load_gather(t_buf, [idx2])
            v = d0 * d0 + d1 * d1 + d2 * d2
            ids = id_buf[pl.ds(g * L, L)]
            cs = plsc.cumsum(v)
            ids_next = _shift_up(ids, iota)
            m_end = (ids != ids_next) | (iota == L - 1)
            m_int = m_end & (iota < L - 1)
            plsc.addupdate_scatter(acc, [ids], cs, mask=m_end)
            plsc.addupdate_scatter(acc, [ids_next], -cs, mask=m_int)
            return gcarry
        lax.fori_loop(0, G, group_body, 0)
        return carry
    lax.fori_loop(0, NCHUNK, chunk_body, 0)

    pltpu.sync_copy(acc, shared.at[s])
    plsc.subcore_barrier()
    pltpu.sync_copy(shared.at[:, pl.ds(s * SEG, SEG)], colbuf)

    def col_body(i, carry):
        tot = colbuf[0, pl.ds(i * L, L)]
        for r in range(1, NS):
            tot = tot + colbuf[r, pl.ds(i * L, L)]
        acc[pl.ds(i * L, L)] = tot
        return carry
    lax.fori_loop(0, SEG // L, col_body, 0)
    pltpu.sync_copy(acc.at[pl.ds(0, SEG)], out_hbm.at[c, pl.ds(s * SEG, SEG)])


@jax.jit
def _run(p_flat, t_flat, ids):
    mesh = plsc.VectorSubcoreMesh(core_axis_name="c", subcore_axis_name="s",
                                  num_cores=NC, num_subcores=NS)
    fn = pl.kernel(
        _sc_body,
        out_type=jax.ShapeDtypeStruct((NC, S), jnp.float32),
        mesh=mesh,
        compiler_params=pltpu.CompilerParams(needs_layout_passes=False),
        scratch_types=[
            pltpu.VMEM((C * 3,), jnp.float32),
            pltpu.VMEM((C * 3,), jnp.float32),
            pltpu.VMEM((C,), jnp.int32),
            pltpu.VMEM((S,), jnp.float32),
            pltpu.VMEM_SHARED((NS, S), jnp.float32),
            pltpu.VMEM((NS, SEG), jnp.float32),
        ],
    )
    return fn(p_flat, t_flat, ids)


def kernel(pred_eps, true_eps, segment_ids):
    p_flat = jnp.reshape(pred_eps, (N * 3,))
    t_flat = jnp.reshape(true_eps, (N * 3,))
    ids = segment_ids.astype(jnp.int32)
    parts = _run(p_flat, t_flat, ids)
    return parts[0] + parts[1]


# speedup vs baseline: 51.6324x; 51.6324x over previous
import functools

import jax
import jax.numpy as jnp
from jax import lax
from jax.experimental import pallas as pl
from jax.experimental.pallas import tpu as pltpu
from jax.experimental.pallas import tpu_sc as plsc

N = 3_200_000
S = 8192
NC, NS, L = 2, 16, 16
NW = NC * NS
R = N // NW
C = 4_000
NCHUNK = R // C
G = C // L
SEG = S // NS

_GATHER_DNUMS = lax.GatherDimensionNumbers(
    offset_dims=(), collapsed_slice_dims=(0,), start_index_map=(0,))


def _shift_up(x, iota):
    idx = jnp.minimum(iota + 1, L - 1)
    return lax.gather(x, idx[:, None], _GATHER_DNUMS, slice_sizes=(1,),
                      mode=lax.GatherScatterMode.PROMISE_IN_BOUNDS)


def _sc_body(p0_hbm, p1_hbm, p2_hbm, t0_hbm, t1_hbm, t2_hbm, ids_hbm,
             out_hbm, p0b, p1b, p2b, t0b, t1b, t2b, id_buf,
             acc, shared, colbuf):
    c = lax.axis_index("c")
    s = lax.axis_index("s")
    wid = c * NS + s
    row0 = wid * R

    iota = lax.iota(jnp.int32, L)
    zeros = jnp.zeros((L,), jnp.float32)

    def zero_body(i, carry):
        acc[pl.ds(i * L, L)] = zeros
        return carry
    lax.fori_loop(0, S // L, zero_body, 0)

    def chunk_body(k, carry):
        base = row0 + k * C
        pltpu.sync_copy(p0_hbm.at[pl.ds(base, C)], p0b)
        pltpu.sync_copy(p1_hbm.at[pl.ds(base, C)], p1b)
        pltpu.sync_copy(p2_hbm.at[pl.ds(base, C)], p2b)
        pltpu.sync_copy(t0_hbm.at[pl.ds(base, C)], t0b)
        pltpu.sync_copy(t1_hbm.at[pl.ds(base, C)], t1b)
        pltpu.sync_copy(t2_hbm.at[pl.ds(base, C)], t2b)
        pltpu.sync_copy(ids_hbm.at[pl.ds(base, C)], id_buf)

        def group_body(g, gcarry):
            o = g * L
            d0 = p0b[pl.ds(o, L)] - t0b[pl.ds(o, L)]
            d1 = p1b[pl.ds(o, L)] - t1b[pl.ds(o, L)]
            d2 = p2b[pl.ds(o, L)] - t2b[pl.ds(o, L)]
            v = d0 * d0 + d1 * d1 + d2 * d2
            ids = id_buf[pl.ds(o, L)]
            cs = plsc.cumsum(v)
            ids_next = _shift_up(ids, iota)
            m_end = (ids != ids_next) | (iota == L - 1)
            m_int = m_end & (iota < L - 1)
            plsc.addupdate_scatter(acc, [ids], cs, mask=m_end)
            plsc.addupdate_scatter(acc, [ids_next], -cs, mask=m_int)
            return gcarry
        lax.fori_loop(0, G, group_body, 0)
        return carry
    lax.fori_loop(0, NCHUNK, chunk_body, 0)

    pltpu.sync_copy(acc, shared.at[s])
    plsc.subcore_barrier()
    pltpu.sync_copy(shared.at[:, pl.ds(s * SEG, SEG)], colbuf)

    def col_body(i, carry):
        tot = colbuf[0, pl.ds(i * L, L)]
        for r in range(1, NS):
            tot = tot + colbuf[r, pl.ds(i * L, L)]
        acc[pl.ds(i * L, L)] = tot
        return carry
    lax.fori_loop(0, SEG // L, col_body, 0)
    pltpu.sync_copy(acc.at[pl.ds(0, SEG)], out_hbm.at[c, pl.ds(s * SEG, SEG)])


@jax.jit
def _run(p0, p1, p2, t0, t1, t2, ids):
    mesh = plsc.VectorSubcoreMesh(core_axis_name="c", subcore_axis_name="s",
                                  num_cores=NC, num_subcores=NS)
    fn = pl.kernel(
        _sc_body,
        out_type=jax.ShapeDtypeStruct((NC, S), jnp.float32),
        mesh=mesh,
        compiler_params=pltpu.CompilerParams(needs_layout_passes=False),
        scratch_types=[
            pltpu.VMEM((C,), jnp.float32),
            pltpu.VMEM((C,), jnp.float32),
            pltpu.VMEM((C,), jnp.float32),
            pltpu.VMEM((C,), jnp.float32),
            pltpu.VMEM((C,), jnp.float32),
            pltpu.VMEM((C,), jnp.float32),
            pltpu.VMEM((C,), jnp.int32),
            pltpu.VMEM((S,), jnp.float32),
            pltpu.VMEM_SHARED((NS, S), jnp.float32),
            pltpu.VMEM((NS, SEG), jnp.float32),
        ],
    )
    return fn(p0, p1, p2, t0, t1, t2, ids)


def kernel(pred_eps, true_eps, segment_ids):
    ids = segment_ids.astype(jnp.int32)
    parts = _run(pred_eps[:, 0], pred_eps[:, 1], pred_eps[:, 2],
                 true_eps[:, 0], true_eps[:, 1], true_eps[:, 2], ids)
    return parts[0] + parts[1]


# speedup vs baseline: 77.7666x; 1.5062x over previous
import functools

import jax
import jax.numpy as jnp
from jax import lax
from jax.experimental import pallas as pl
from jax.experimental.pallas import tpu as pltpu
from jax.experimental.pallas import tpu_sc as plsc

N = 3_200_000
S = 8192
NC, NS, L = 2, 16, 16
NW = NC * NS
R = N // NW
C = 4_000
NCHUNK = R // C
G = C // L
SEG = S // NS

_GATHER_DNUMS = lax.GatherDimensionNumbers(
    offset_dims=(), collapsed_slice_dims=(0,), start_index_map=(0,))


def _shift_up(x, iota):
    idx = jnp.minimum(iota + 1, L - 1)
    return lax.gather(x, idx[:, None], _GATHER_DNUMS, slice_sizes=(1,),
                      mode=lax.GatherScatterMode.PROMISE_IN_BOUNDS)


def _sc_body(p0_hbm, p1_hbm, p2_hbm, t0_hbm, t1_hbm, t2_hbm, ids_hbm,
             out_hbm,
             p0b0, p1b0, p2b0, t0b0, t1b0, t2b0, idb0,
             p0b1, p1b1, p2b1, t0b1, t1b1, t2b1, idb1,
             acc, shared, colbuf, sem0, sem1):
    c = lax.axis_index("c")
    s = lax.axis_index("s")
    wid = c * NS + s
    row0 = wid * R

    hbms = (p0_hbm, p1_hbm, p2_hbm, t0_hbm, t1_hbm, t2_hbm, ids_hbm)
    slot_bufs = ((p0b0, p1b0, p2b0, t0b0, t1b0, t2b0, idb0),
                 (p0b1, p1b1, p2b1, t0b1, t1b1, t2b1, idb1))
    sems = (sem0, sem1)

    iota = lax.iota(jnp.int32, L)
    zeros = jnp.zeros((L,), jnp.float32)
    lane_last = iota == L - 1
    lane_not_last = iota < L - 1

    def zero_body(i, carry):
        acc[pl.ds(i * L, L)] = zeros
        return carry
    lax.fori_loop(0, S // L, zero_body, 0)

    def start_chunk(k, slot):
        base = row0 + k * C
        for hbm, buf in zip(hbms, slot_bufs[slot]):
            pltpu.async_copy(hbm.at[pl.ds(base, C)], buf, sems[slot])

    def wait_chunk(k, slot):
        base = row0 + k * C
        for hbm, buf in zip(hbms, slot_bufs[slot]):
            pltpu.make_async_copy(hbm.at[pl.ds(base, C)], buf,
                                  sems[slot]).wait()

    def compute_chunk(slot):
        p0s, p1s, p2s, t0s, t1s, t2s, idss = slot_bufs[slot]

        def group_body(g, gcarry):
            o = g * L
            d0 = p0s[pl.ds(o, L)] - t0s[pl.ds(o, L)]
            d1 = p1s[pl.ds(o, L)] - t1s[pl.ds(o, L)]
            d2 = p2s[pl.ds(o, L)] - t2s[pl.ds(o, L)]
            v = d0 * d0 + d1 * d1 + d2 * d2
            ids = idss[pl.ds(o, L)]
            cs = plsc.cumsum(v)
            ids_next = _shift_up(ids, iota)
            m_end = (ids != ids_next) | lane_last
            m_int = m_end & lane_not_last
            plsc.addupdate_scatter(acc, [ids], cs, mask=m_end)
            plsc.addupdate_scatter(acc, [ids_next], -cs, mask=m_int)
            return gcarry
        lax.fori_loop(0, G, group_body, 0)

    start_chunk(0, 0)
    start_chunk(1, 1)

    def pair_body(j, carry):
        k0 = 2 * j
        wait_chunk(k0, 0)
        compute_chunk(0)

        @pl.when(k0 + 2 < NCHUNK)
        def _():
            start_chunk(k0 + 2, 0)

        @pl.when(k0 + 1 < NCHUNK)
        def _():
            wait_chunk(k0 + 1, 1)
            compute_chunk(1)

            @pl.when(k0 + 3 < NCHUNK)
            def _():
                start_chunk(k0 + 3, 1)
        return carry
    lax.fori_loop(0, (NCHUNK + 1) // 2, pair_body, 0)

    pltpu.sync_copy(acc, shared.at[s])
    plsc.subcore_barrier()
    pltpu.sync_copy(shared.at[:, pl.ds(s * SEG, SEG)], colbuf)

    def col_body(i, carry):
        tot = colbuf[0, pl.ds(i * L, L)]
        for r in range(1, NS):
            tot = tot + colbuf[r, pl.ds(i * L, L)]
        acc[pl.ds(i * L, L)] = tot
        return carry
    lax.fori_loop(0, SEG // L, col_body, 0)
    pltpu.sync_copy(acc.at[pl.ds(0, SEG)], out_hbm.at[c, pl.ds(s * SEG, SEG)])


@jax.jit
def _run(p0, p1, p2, t0, t1, t2, ids):
    mesh = plsc.VectorSubcoreMesh(core_axis_name="c", subcore_axis_name="s",
                                  num_cores=NC, num_subcores=NS)
    fn = pl.kernel(
        _sc_body,
        out_type=jax.ShapeDtypeStruct((NC, S), jnp.float32),
        mesh=mesh,
        compiler_params=pltpu.CompilerParams(needs_layout_passes=False),
        scratch_types=[
            pltpu.VMEM((C,), jnp.float32),
            pltpu.VMEM((C,), jnp.float32),
            pltpu.VMEM((C,), jnp.float32),
            pltpu.VMEM((C,), jnp.float32),
            pltpu.VMEM((C,), jnp.float32),
            pltpu.VMEM((C,), jnp.float32),
            pltpu.VMEM((C,), jnp.int32),
            pltpu.VMEM((C,), jnp.float32),
            pltpu.VMEM((C,), jnp.float32),
            pltpu.VMEM((C,), jnp.float32),
            pltpu.VMEM((C,), jnp.float32),
            pltpu.VMEM((C,), jnp.float32),
            pltpu.VMEM((C,), jnp.float32),
            pltpu.VMEM((C,), jnp.int32),
            pltpu.VMEM((S,), jnp.float32),
            pltpu.VMEM_SHARED((NS, S), jnp.float32),
            pltpu.VMEM((NS, SEG), jnp.float32),
            pltpu.SemaphoreType.DMA,
            pltpu.SemaphoreType.DMA,
        ],
    )
    return fn(p0, p1, p2, t0, t1, t2, ids)


def kernel(pred_eps, true_eps, segment_ids):
    ids = segment_ids.astype(jnp.int32)
    parts = _run(pred_eps[:, 0], pred_eps[:, 1], pred_eps[:, 2],
                 true_eps[:, 0], true_eps[:, 1], true_eps[:, 2], ids)
    return parts[0] + parts[1]


# speedup vs baseline: 111.2284x; 1.4303x over previous
import functools

import jax
import jax.numpy as jnp
from jax import lax
from jax.experimental import pallas as pl
from jax.experimental.pallas import tpu as pltpu
from jax.experimental.pallas import tpu_sc as plsc

N = 3_200_000
S = 8192
NC, NS, L = 2, 16, 16
NW = NC * NS
R = N // NW
C = 4_000
NCHUNK = R // C
G = C // L
SEG = S // NS

TC_BLK = 25_600

_GATHER_DNUMS = lax.GatherDimensionNumbers(
    offset_dims=(), collapsed_slice_dims=(0,), start_index_map=(0,))


def _shift_up(x, iota):
    idx = jnp.minimum(iota + 1, L - 1)
    return lax.gather(x, idx[:, None], _GATHER_DNUMS, slice_sizes=(1,),
                      mode=lax.GatherScatterMode.PROMISE_IN_BOUNDS)


def _rowerr_body(p_ref, t_ref, v_ref):
    d = p_ref[...] - t_ref[...]
    e = d * d
    v_ref[...] = e[0, :] + e[1, :] + e[2, :]


@jax.jit
def _rowerr(pT, tT):
    return pl.pallas_call(
        _rowerr_body,
        grid=(N // TC_BLK,),
        in_specs=[pl.BlockSpec((3, TC_BLK), lambda i: (0, i)),
                  pl.BlockSpec((3, TC_BLK), lambda i: (0, i))],
        out_specs=pl.BlockSpec((TC_BLK,), lambda i: (i,)),
        out_shape=jax.ShapeDtypeStruct((N,), jnp.float32),
    )(pT, tT)


def _sc_body(v_hbm, ids_hbm, out_hbm,
             vb0, idb0, vb1, idb1,
             acc, shared, colbuf, sem0, sem1):
    c = lax.axis_index("c")
    s = lax.axis_index("s")
    wid = c * NS + s
    row0 = wid * R

    hbms = (v_hbm, ids_hbm)
    slot_bufs = ((vb0, idb0), (vb1, idb1))
    sems = (sem0, sem1)

    iota = lax.iota(jnp.int32, L)
    zeros = jnp.zeros((L,), jnp.float32)
    lane_last = iota == L - 1
    lane_not_last = iota < L - 1

    def zero_body(i, carry):
        acc[pl.ds(i * L, L)] = zeros
        return carry
    lax.fori_loop(0, S // L, zero_body, 0)

    def start_chunk(k, slot):
        base = row0 + k * C
        for hbm, buf in zip(hbms, slot_bufs[slot]):
            pltpu.async_copy(hbm.at[pl.ds(base, C)], buf, sems[slot])

    def wait_chunk(k, slot):
        base = row0 + k * C
        for hbm, buf in zip(hbms, slot_bufs[slot]):
            pltpu.make_async_copy(hbm.at[pl.ds(base, C)], buf,
                                  sems[slot]).wait()

    def compute_chunk(slot):
        vs, idss = slot_bufs[slot]

        def group_body(g, gcarry):
            o = g * L
            v = vs[pl.ds(o, L)]
            ids = idss[pl.ds(o, L)]
            cs = plsc.cumsum(v)
            ids_next = _shift_up(ids, iota)
            m_end = (ids != ids_next) | lane_last
            m_int = m_end & lane_not_last
            plsc.addupdate_scatter(acc, [ids], cs, mask=m_end)
            plsc.addupdate_scatter(acc, [ids_next], -cs, mask=m_int)
            return gcarry
        lax.fori_loop(0, G, group_body, 0)

    start_chunk(0, 0)
    start_chunk(1, 1)

    def pair_body(j, carry):
        k0 = 2 * j
        wait_chunk(k0, 0)
        compute_chunk(0)

        @pl.when(k0 + 2 < NCHUNK)
        def _():
            start_chunk(k0 + 2, 0)

        @pl.when(k0 + 1 < NCHUNK)
        def _():
            wait_chunk(k0 + 1, 1)
            compute_chunk(1)

            @pl.when(k0 + 3 < NCHUNK)
            def _():
                start_chunk(k0 + 3, 1)
        return carry
    lax.fori_loop(0, (NCHUNK + 1) // 2, pair_body, 0)

    pltpu.sync_copy(acc, shared.at[s])
    plsc.subcore_barrier()
    pltpu.sync_copy(shared.at[:, pl.ds(s * SEG, SEG)], colbuf)

    def col_body(i, carry):
        tot = colbuf[0, pl.ds(i * L, L)]
        for r in range(1, NS):
            tot = tot + colbuf[r, pl.ds(i * L, L)]
        acc[pl.ds(i * L, L)] = tot
        return carry
    lax.fori_loop(0, SEG // L, col_body, 0)
    pltpu.sync_copy(acc.at[pl.ds(0, SEG)], out_hbm.at[c, pl.ds(s * SEG, SEG)])


@jax.jit
def _run(v, ids):
    mesh = plsc.VectorSubcoreMesh(core_axis_name="c", subcore_axis_name="s",
                                  num_cores=NC, num_subcores=NS)
    fn = pl.kernel(
        _sc_body,
        out_type=jax.ShapeDtypeStruct((NC, S), jnp.float32),
        mesh=mesh,
        compiler_params=pltpu.CompilerParams(needs_layout_passes=False),
        scratch_types=[
            pltpu.VMEM((C,), jnp.float32),
            pltpu.VMEM((C,), jnp.int32),
            pltpu.VMEM((C,), jnp.float32),
            pltpu.VMEM((C,), jnp.int32),
            pltpu.VMEM((S,), jnp.float32),
            pltpu.VMEM_SHARED((NS, S), jnp.float32),
            pltpu.VMEM((NS, SEG), jnp.float32),
            pltpu.SemaphoreType.DMA,
            pltpu.SemaphoreType.DMA,
        ],
    )
    return fn(v, ids)


def kernel(pred_eps, true_eps, segment_ids):
    ids = segment_ids.astype(jnp.int32)
    v = _rowerr(jnp.swapaxes(pred_eps, 0, 1), jnp.swapaxes(true_eps, 0, 1))
    parts = _run(v, ids)
    return parts[0] + parts[1]


# speedup vs baseline: 151.5875x; 1.3628x over previous
import functools

import jax
import jax.numpy as jnp
from jax import lax
from jax.experimental import pallas as pl
from jax.experimental.pallas import tpu as pltpu
from jax.experimental.pallas import tpu_sc as plsc

N = 3_200_000
S = 8192
NC, NS, L = 2, 16, 16
NW = NC * NS
R = N // NW
C = 4_000
NCHUNK = R // C
G = C // L
SEG = S // NS

TC_BLK = 128_000

_GATHER_DNUMS = lax.GatherDimensionNumbers(
    offset_dims=(), collapsed_slice_dims=(0,), start_index_map=(0,))


def _shift_up(x, iota):
    idx = jnp.minimum(iota + 1, L - 1)
    return lax.gather(x, idx[:, None], _GATHER_DNUMS, slice_sizes=(1,),
                      mode=lax.GatherScatterMode.PROMISE_IN_BOUNDS)


def _rowerr_body(p_ref, t_ref, v_ref):
    d = p_ref[...] - t_ref[...]
    e = d * d
    v_ref[...] = e[0, :] + e[1, :] + e[2, :]


@jax.jit
def _rowerr(pT, tT):
    return pl.pallas_call(
        _rowerr_body,
        grid=(N // TC_BLK,),
        in_specs=[pl.BlockSpec((3, TC_BLK), lambda i: (0, i)),
                  pl.BlockSpec((3, TC_BLK), lambda i: (0, i))],
        out_specs=pl.BlockSpec((TC_BLK,), lambda i: (i,)),
        out_shape=jax.ShapeDtypeStruct((N,), jnp.float32),
        compiler_params=pltpu.CompilerParams(
            dimension_semantics=("arbitrary",)),
    )(pT, tT)


def _sc_body(v_hbm, ids_hbm, out_hbm,
             vb0, idb0, vb1, idb1,
             acc, shared, colbuf, sem0, sem1):
    c = lax.axis_index("c")
    s = lax.axis_index("s")
    wid = c * NS + s
    row0 = wid * R

    hbms = (v_hbm, ids_hbm)
    slot_bufs = ((vb0, idb0), (vb1, idb1))
    sems = (sem0, sem1)

    iota = lax.iota(jnp.int32, L)
    zeros = jnp.zeros((L,), jnp.float32)
    lane_last = iota == L - 1
    lane_not_last = iota < L - 1

    def zero_body(i, carry):
        acc[pl.ds(i * L, L)] = zeros
        return carry
    lax.fori_loop(0, S // L, zero_body, 0)

    def start_chunk(k, slot):
        base = row0 + k * C
        for hbm, buf in zip(hbms, slot_bufs[slot]):
            pltpu.async_copy(hbm.at[pl.ds(base, C)], buf, sems[slot])

    def wait_chunk(k, slot):
        base = row0 + k * C
        for hbm, buf in zip(hbms, slot_bufs[slot]):
            pltpu.make_async_copy(hbm.at[pl.ds(base, C)], buf,
                                  sems[slot]).wait()

    def compute_chunk(slot):
        vs, idss = slot_bufs[slot]

        def group_body(g, gcarry):
            o = g * L
            v = vs[pl.ds(o, L)]
            ids = idss[pl.ds(o, L)]
            cs = plsc.cumsum(v)
            ids_next = _shift_up(ids, iota)
            m_end = (ids != ids_next) | lane_last
            m_int = m_end & lane_not_last
            plsc.addupdate_scatter(acc, [ids], cs, mask=m_end)
            plsc.addupdate_scatter(acc, [ids_next], -cs, mask=m_int)
            return gcarry
        lax.fori_loop(0, G, group_body, 0)

    start_chunk(0, 0)
    start_chunk(1, 1)

    def pair_body(j, carry):
        k0 = 2 * j
        wait_chunk(k0, 0)
        compute_chunk(0)

        @pl.when(k0 + 2 < NCHUNK)
        def _():
            start_chunk(k0 + 2, 0)

        @pl.when(k0 + 1 < NCHUNK)
        def _():
            wait_chunk(k0 + 1, 1)
            compute_chunk(1)

            @pl.when(k0 + 3 < NCHUNK)
            def _():
                start_chunk(k0 + 3, 1)
        return carry
    lax.fori_loop(0, (NCHUNK + 1) // 2, pair_body, 0)

    pltpu.sync_copy(acc, shared.at[s])
    plsc.subcore_barrier()
    pltpu.sync_copy(shared.at[:, pl.ds(s * SEG, SEG)], colbuf)

    def col_body(i, carry):
        tot = colbuf[0, pl.ds(i * L, L)]
        for r in range(1, NS):
            tot = tot + colbuf[r, pl.ds(i * L, L)]
        acc[pl.ds(i * L, L)] = tot
        return carry
    lax.fori_loop(0, SEG // L, col_body, 0)
    pltpu.sync_copy(acc.at[pl.ds(0, SEG)], out_hbm.at[c, pl.ds(s * SEG, SEG)])


@jax.jit
def _run(v, ids):
    mesh = plsc.VectorSubcoreMesh(core_axis_name="c", subcore_axis_name="s",
                                  num_cores=NC, num_subcores=NS)
    fn = pl.kernel(
        _sc_body,
        out_type=jax.ShapeDtypeStruct((NC, S), jnp.float32),
        mesh=mesh,
        compiler_params=pltpu.CompilerParams(needs_layout_passes=False),
        scratch_types=[
            pltpu.VMEM((C,), jnp.float32),
            pltpu.VMEM((C,), jnp.int32),
            pltpu.VMEM((C,), jnp.float32),
            pltpu.VMEM((C,), jnp.int32),
            pltpu.VMEM((S,), jnp.float32),
            pltpu.VMEM_SHARED((NS, S), jnp.float32),
            pltpu.VMEM((NS, SEG), jnp.float32),
            pltpu.SemaphoreType.DMA,
            pltpu.SemaphoreType.DMA,
        ],
    )
    return fn(v, ids)


def kernel(pred_eps, true_eps, segment_ids):
    ids = segment_ids.astype(jnp.int32)
    v = _rowerr(jnp.swapaxes(pred_eps, 0, 1), jnp.swapaxes(true_eps, 0, 1))
    parts = _run(v, ids)
    return parts[0] + parts[1]


# speedup vs baseline: 154.5355x; 1.0194x over previous
import functools

import jax
import jax.numpy as jnp
from jax import lax
from jax.experimental import pallas as pl
from jax.experimental.pallas import tpu as pltpu
from jax.experimental.pallas import tpu_sc as plsc

N = 3_200_000
S = 8192
NC, NS, L = 2, 16, 16
NW = NC * NS
R = N // NW
C = 4_000
NCHUNK = R // C
G = C // L
SEG = S // NS

TC_BLK = 640_000

_GATHER_DNUMS = lax.GatherDimensionNumbers(
    offset_dims=(), collapsed_slice_dims=(0,), start_index_map=(0,))


def _shift_up(x, iota):
    idx = jnp.minimum(iota + 1, L - 1)
    return lax.gather(x, idx[:, None], _GATHER_DNUMS, slice_sizes=(1,),
                      mode=lax.GatherScatterMode.PROMISE_IN_BOUNDS)


def _rowerr_body(p_ref, t_ref, v_ref):
    d = p_ref[...] - t_ref[...]
    e = d * d
    v_ref[...] = e[0, :] + e[1, :] + e[2, :]


@jax.jit
def _rowerr(pT, tT):
    return pl.pallas_call(
        _rowerr_body,
        grid=(N // TC_BLK,),
        in_specs=[pl.BlockSpec((3, TC_BLK), lambda i: (0, i)),
                  pl.BlockSpec((3, TC_BLK), lambda i: (0, i))],
        out_specs=pl.BlockSpec((TC_BLK,), lambda i: (i,)),
        out_shape=jax.ShapeDtypeStruct((N,), jnp.float32),
        compiler_params=pltpu.CompilerParams(
            dimension_semantics=("arbitrary",)),
    )(pT, tT)


def _sc_body(v_hbm, ids_hbm, out_hbm,
             vb0, idb0, vb1, idb1,
             acc, shared, colbuf, sem0, sem1):
    c = lax.axis_index("c")
    s = lax.axis_index("s")
    wid = c * NS + s
    row0 = wid * R

    hbms = (v_hbm, ids_hbm)
    slot_bufs = ((vb0, idb0), (vb1, idb1))
    sems = (sem0, sem1)

    iota = lax.iota(jnp.int32, L)
    zeros = jnp.zeros((L,), jnp.float32)
    lane_last = iota == L - 1
    lane_not_last = iota < L - 1

    def zero_body(i, carry):
        acc[pl.ds(i * L, L)] = zeros
        return carry
    lax.fori_loop(0, S // L, zero_body, 0)

    def start_chunk(k, slot):
        base = row0 + k * C
        for hbm, buf in zip(hbms, slot_bufs[slot]):
            pltpu.async_copy(hbm.at[pl.ds(base, C)], buf, sems[slot])

    def wait_chunk(k, slot):
        base = row0 + k * C
        for hbm, buf in zip(hbms, slot_bufs[slot]):
            pltpu.make_async_copy(hbm.at[pl.ds(base, C)], buf,
                                  sems[slot]).wait()

    def compute_chunk(slot):
        vs, idss = slot_bufs[slot]
        UNROLL = 5

        def group_body(g, gcarry):
            o0 = g * (L * UNROLL)
            flushes = []
            for u in range(UNROLL):
                o = o0 + u * L
                v = vs[pl.ds(o, L)]
                ids = idss[pl.ds(o, L)]
                cs = plsc.cumsum(v)
                ids_next = _shift_up(ids, iota)
                m_end = (ids != ids_next) | lane_last
                m_int = m_end & lane_not_last
                flushes.append((ids, cs, m_end, ids_next, m_int))
            for ids, cs, m_end, ids_next, m_int in flushes:
                plsc.addupdate_scatter(acc, [ids], cs, mask=m_end)
                plsc.addupdate_scatter(acc, [ids_next], -cs, mask=m_int)
            return gcarry
        lax.fori_loop(0, G // UNROLL, group_body, 0)

    start_chunk(0, 0)
    start_chunk(1, 1)

    def pair_body(j, carry):
        k0 = 2 * j
        wait_chunk(k0, 0)
        compute_chunk(0)

        @pl.when(k0 + 2 < NCHUNK)
        def _():
            start_chunk(k0 + 2, 0)

        @pl.when(k0 + 1 < NCHUNK)
        def _():
            wait_chunk(k0 + 1, 1)
            compute_chunk(1)

            @pl.when(k0 + 3 < NCHUNK)
            def _():
                start_chunk(k0 + 3, 1)
        return carry
    lax.fori_loop(0, (NCHUNK + 1) // 2, pair_body, 0)

    pltpu.sync_copy(acc, shared.at[s])
    plsc.subcore_barrier()
    pltpu.sync_copy(shared.at[:, pl.ds(s * SEG, SEG)], colbuf)

    def col_body(i, carry):
        tot = colbuf[0, pl.ds(i * L, L)]
        for r in range(1, NS):
            tot = tot + colbuf[r, pl.ds(i * L, L)]
        acc[pl.ds(i * L, L)] = tot
        return carry
    lax.fori_loop(0, SEG // L, col_body, 0)
    pltpu.sync_copy(acc.at[pl.ds(0, SEG)], out_hbm.at[c, pl.ds(s * SEG, SEG)])


@jax.jit
def _run(v, ids):
    mesh = plsc.VectorSubcoreMesh(core_axis_name="c", subcore_axis_name="s",
                                  num_cores=NC, num_subcores=NS)
    fn = pl.kernel(
        _sc_body,
        out_type=jax.ShapeDtypeStruct((NC, S), jnp.float32),
        mesh=mesh,
        compiler_params=pltpu.CompilerParams(needs_layout_passes=False),
        scratch_types=[
            pltpu.VMEM((C,), jnp.float32),
            pltpu.VMEM((C,), jnp.int32),
            pltpu.VMEM((C,), jnp.float32),
            pltpu.VMEM((C,), jnp.int32),
            pltpu.VMEM((S,), jnp.float32),
            pltpu.VMEM_SHARED((NS, S), jnp.float32),
            pltpu.VMEM((NS, SEG), jnp.float32),
            pltpu.SemaphoreType.DMA,
            pltpu.SemaphoreType.DMA,
        ],
    )
    return fn(v, ids)


def kernel(pred_eps, true_eps, segment_ids):
    ids = segment_ids.astype(jnp.int32)
    v = _rowerr(jnp.swapaxes(pred_eps, 0, 1), jnp.swapaxes(true_eps, 0, 1))
    parts = _run(v, ids)
    return parts[0] + parts[1]


# speedup vs baseline: 228.5007x; 1.4786x over previous
import functools

import jax
import jax.numpy as jnp
from jax import lax
from jax.experimental import pallas as pl
from jax.experimental.pallas import tpu as pltpu
from jax.experimental.pallas import tpu_sc as plsc

N = 3_200_000
S = 8192
NC, NS, L = 2, 16, 16
NW = NC * NS
C = 4_000
SEG = S // NS
TC_BLK = 128_000

PIECES = ((0, 2_048_000), (2_048_000, 1_152_000))

_GATHER_DNUMS = lax.GatherDimensionNumbers(
    offset_dims=(), collapsed_slice_dims=(0,), start_index_map=(0,))


def _shift_up(x, iota):
    idx = jnp.minimum(iota + 1, L - 1)
    return lax.gather(x, idx[:, None], _GATHER_DNUMS, slice_sizes=(1,),
                      mode=lax.GatherScatterMode.PROMISE_IN_BOUNDS)


def _rowerr_body(p_ref, t_ref, v_ref):
    d = p_ref[...] - t_ref[...]
    e = d * d
    v_ref[...] = e[0, :] + e[1, :] + e[2, :]


def _make_rowerr(off, size):
    ob = off // TC_BLK

    @jax.jit
    def _rowerr(pT, tT):
        return pl.pallas_call(
            _rowerr_body,
            grid=(size // TC_BLK,),
            in_specs=[pl.BlockSpec((3, TC_BLK), lambda i: (0, i + ob)),
                      pl.BlockSpec((3, TC_BLK), lambda i: (0, i + ob))],
            out_specs=pl.BlockSpec((TC_BLK,), lambda i: (i,)),
            out_shape=jax.ShapeDtypeStruct((size,), jnp.float32),
            compiler_params=pltpu.CompilerParams(
                dimension_semantics=("arbitrary",)),
        )(pT, tT)
    return _rowerr


def _make_sc_body(off, size):
    rpw = size // NW
    nchunk = rpw // C
    G = C // L

    def _sc_body(v_hbm, ids_hbm, out_hbm,
                 vb0, idb0, vb1, idb1,
                 acc, shared, colbuf, sem0, sem1):
        c = lax.axis_index("c")
        s = lax.axis_index("s")
        wid = c * NS + s
        row0 = wid * rpw

        hbms = (v_hbm, ids_hbm)
        offs = (0, off)
        slot_bufs = ((vb0, idb0), (vb1, idb1))
        sems = (sem0, sem1)

        iota = lax.iota(jnp.int32, L)
        zeros = jnp.zeros((L,), jnp.float32)
        lane_last = iota == L - 1
        lane_not_last = iota < L - 1

        def zero_body(i, carry):
            acc[pl.ds(i * L, L)] = zeros
            return carry
        lax.fori_loop(0, S // L, zero_body, 0)

        def start_chunk(k, slot):
            base = row0 + k * C
            for hbm, o, buf in zip(hbms, offs, slot_bufs[slot]):
                pltpu.async_copy(hbm.at[pl.ds(o + base, C)], buf, sems[slot])

        def wait_chunk(k, slot):
            base = row0 + k * C
            for hbm, o, buf in zip(hbms, offs, slot_bufs[slot]):
                pltpu.make_async_copy(hbm.at[pl.ds(o + base, C)], buf,
                                      sems[slot]).wait()

        def compute_chunk(slot):
            vs, idss = slot_bufs[slot]
            UNROLL = 5

            def group_body(g, gcarry):
                o0 = g * (L * UNROLL)
                flushes = []
                for u in range(UNROLL):
                    o = o0 + u * L
                    v = vs[pl.ds(o, L)]
                    ids = idss[pl.ds(o, L)]
                    cs = plsc.cumsum(v)
                    ids_next = _shift_up(ids, iota)
                    m_end = (ids != ids_next) | lane_last
                    m_int = m_end & lane_not_last
                    flushes.append((ids, cs, m_end, ids_next, m_int))
                for ids, cs, m_end, ids_next, m_int in flushes:
                    plsc.addupdate_scatter(acc, [ids], cs, mask=m_end)
                    plsc.addupdate_scatter(acc, [ids_next], -cs, mask=m_int)
                return gcarry
            lax.fori_loop(0, G // UNROLL, group_body, 0)

        start_chunk(0, 0)
        start_chunk(1, 1)

        def pair_body(j, carry):
            k0 = 2 * j
            wait_chunk(k0, 0)
            compute_chunk(0)

            @pl.when(k0 + 2 < nchunk)
            def _():
                start_chunk(k0 + 2, 0)

            @pl.when(k0 + 1 < nchunk)
            def _():
                wait_chunk(k0 + 1, 1)
                compute_chunk(1)

                @pl.when(k0 + 3 < nchunk)
                def _():
                    start_chunk(k0 + 3, 1)
            return carry
        lax.fori_loop(0, (nchunk + 1) // 2, pair_body, 0)

        pltpu.sync_copy(acc, shared.at[s])
        plsc.subcore_barrier()
        pltpu.sync_copy(shared.at[:, pl.ds(s * SEG, SEG)], colbuf)

        def col_body(i, carry):
            tot = colbuf[0, pl.ds(i * L, L)]
            for r in range(1, NS):
                tot = tot + colbuf[r, pl.ds(i * L, L)]
            acc[pl.ds(i * L, L)] = tot
            return carry
        lax.fori_loop(0, SEG // L, col_body, 0)
        pltpu.sync_copy(acc.at[pl.ds(0, SEG)],
                        out_hbm.at[c, pl.ds(s * SEG, SEG)])

    return _sc_body


def _make_run(off, size):
    body = _make_sc_body(off, size)

    @jax.jit
    def _run(v, ids):
        mesh = plsc.VectorSubcoreMesh(core_axis_name="c",
                                      subcore_axis_name="s",
                                      num_cores=NC, num_subcores=NS)
        fn = pl.kernel(
            body,
            out_type=jax.ShapeDtypeStruct((NC, S), jnp.float32),
            mesh=mesh,
            compiler_params=pltpu.CompilerParams(needs_layout_passes=False),
            scratch_types=[
                pltpu.VMEM((C,), jnp.float32),
                pltpu.VMEM((C,), jnp.int32),
                pltpu.VMEM((C,), jnp.float32),
                pltpu.VMEM((C,), jnp.int32),
                pltpu.VMEM((S,), jnp.float32),
                pltpu.VMEM_SHARED((NS, S), jnp.float32),
                pltpu.VMEM((NS, SEG), jnp.float32),
                pltpu.SemaphoreType.DMA,
                pltpu.SemaphoreType.DMA,
            ],
        )
        return fn(v, ids)
    return _run


_rowerr_fns = [_make_rowerr(o, sz) for o, sz in PIECES]
_run_fns = [_make_run(o, sz) for o, sz in PIECES]


def kernel(pred_eps, true_eps, segment_ids):
    ids = segment_ids.astype(jnp.int32)
    pT = jnp.swapaxes(pred_eps, 0, 1)
    tT = jnp.swapaxes(true_eps, 0, 1)
    out = None
    for re_fn, run_fn in zip(_rowerr_fns, _run_fns):
        v = re_fn(pT, tT)
        parts = run_fn(v, ids)
        piece = parts[0] + parts[1]
        out = piece if out is None else out + piece
    return out
